# baseline (speedup 1.0000x reference)
import functools

import jax
import jax.numpy as jnp
from jax import lax
from jax.experimental import pallas as pl
from jax.experimental.pallas import tpu as pltpu
from jax.experimental.pallas import tpu_sc as plsc

B = 256
N_IN = 64
N_OUT = 256
H = 1024
F = 8192
C = 32768

NC = 2
NS = 16
NW = NC * NS
RPW = B // NW
L = 16

_MESH = plsc.VectorSubcoreMesh(
    core_axis_name="c", subcore_axis_name="s", num_cores=NC, num_subcores=NS)


def _wid():
    return lax.axis_index("s") * NC + lax.axis_index("c")


@functools.partial(
    pl.kernel,
    out_type=jax.ShapeDtypeStruct((B, F), jnp.float32),
    mesh=_MESH,
    compiler_params=pltpu.CompilerParams(needs_layout_passes=False),
    scratch_types=[
        pltpu.VMEM((RPW * F,), jnp.float32),
        pltpu.VMEM((RPW * N_IN,), jnp.int32),
        pltpu.VMEM((RPW * N_IN,), jnp.float32),
    ],
)
def _build_x(vals_hbm, idx_hbm, zeros_hbm, x_hbm, xbuf, idxv, valv):
    w = _wid()
    b0 = w * RPW
    pltpu.sync_copy(zeros_hbm, xbuf)
    pltpu.sync_copy(idx_hbm.at[pl.ds(b0 * N_IN, RPW * N_IN)], idxv)
    pltpu.sync_copy(vals_hbm.at[pl.ds(b0 * N_IN, RPW * N_IN)], valv)
    lane = lax.iota(jnp.int32, L)
    masks = [lane == l for l in range(L)]
    for r in range(RPW):
        for g in range(N_IN // L):
            ig = idxv[pl.ds(r * N_IN + g * L, L)] + (r * F)
            vg = valv[pl.ds(r * N_IN + g * L, L)]
            for l in range(L):
                plsc.addupdate_scatter(xbuf, [ig], vg, mask=masks[l])
    for r in range(RPW):
        pltpu.sync_copy(xbuf.at[pl.ds(r * F, F)], x_hbm.at[b0 + r])


_KBLK = F // 8


def _mm_body(x_ref, w_ref, b_ref, o_ref):
    k = pl.program_id(0)

    @pl.when(k == 0)
    def _init():
        o_ref[...] = jnp.zeros_like(o_ref)

    o_ref[...] += lax.dot_general(
        x_ref[...], w_ref[...], (((1,), (1,)), ((), ())),
        preferred_element_type=jnp.float32)

    @pl.when(k == pl.num_programs(0) - 1)
    def _fin():
        o_ref[...] = jnp.maximum(o_ref[...] + b_ref[...], 0.0)


def _layer1(x, w1, b1):
    return pl.pallas_call(
        _mm_body,
        grid=(F // _KBLK,),
        in_specs=[
            pl.BlockSpec((B, _KBLK), lambda k: (0, k)),
            pl.BlockSpec((H, _KBLK), lambda k: (0, k)),
            pl.BlockSpec((1, H), lambda k: (0, 0)),
        ],
        out_specs=pl.BlockSpec((B, H), lambda k: (0, 0)),
        out_shape=jax.ShapeDtypeStruct((B, H), jnp.float32),
    )(x, w1, b1[None, :])


KROWS = 32
NBUF = 3
NCHUNK = RPW * N_OUT // KROWS
CPB = N_OUT // KROWS


_UNROLL = 4


def _dot16(rows, row_off, v1row, tr, lane):
    def jstep(j, accs):
        for u in range(_UNROLL):
            jj = j * _UNROLL + u
            v1 = v1row[pl.ds(jj * L, L)]
            accs = tuple(accs[o] + rows[row_off + o, pl.ds(jj * L, L)] * v1
                         for o in range(L))
        return accs

    accs = lax.fori_loop(
        0, H // L // _UNROLL, jstep,
        tuple(jnp.zeros((L,), jnp.float32) for _ in range(L)))
    for o in range(L):
        tr[pl.ds(o * L, L)] = accs[o]
    tot = jnp.zeros((L,), jnp.float32)
    for l in range(L):
        col = plsc.load_gather(tr, [lane * L + l])
        tot = tot + col
    return tot


@functools.partial(
    pl.kernel,
    out_type=jax.ShapeDtypeStruct((B, N_OUT), jnp.float32),
    mesh=_MESH,
    compiler_params=pltpu.CompilerParams(needs_layout_passes=False),
    scratch_types=[
        pltpu.VMEM((RPW * N_OUT,), jnp.int32),
        pltpu.VMEM((H,), jnp.float32),
        pltpu.VMEM((NBUF, KROWS, H), jnp.float32),
        pltpu.VMEM((RPW * N_OUT,), jnp.float32),
        pltpu.VMEM((L * L,), jnp.float32),
    ] + [pltpu.SemaphoreType.DMA] * NBUF,
)
def _layer2(v1_hbm, idx_hbm, w2_hbm, b2_hbm, out_hbm,
            idxv, v1row, ring, outv, tr, *sems):
    w = _wid()
    b0 = w * RPW
    lane = lax.iota(jnp.int32, L)
    pltpu.sync_copy(idx_hbm.at[pl.ds(b0 * N_OUT, RPW * N_OUT)], idxv)
    for h in range(RPW * N_OUT // 128):
        pltpu.sync_copy(b2_hbm.at[idxv.at[pl.ds(h * 128, 128)]],
                        outv.at[pl.ds(h * 128, 128)])

    def chunk_idx(t):
        return idxv.at[pl.ds(t * KROWS, KROWS)]

    def issue(t, k):
        return pltpu.async_copy(w2_hbm.at[chunk_idx(t)], ring.at[k], sems[k])

    def wait(t, k):
        pltpu.make_async_copy(w2_hbm.at[chunk_idx(t)], ring.at[k],
                              sems[k]).wait()

    def compute(t, k):
        @pl.when(t % CPB == 0)
        def _refresh():
            pltpu.sync_copy(v1_hbm.at[b0 + t // CPB], v1row)
        for half in range(KROWS // L):
            tot = _dot16(ring.at[k], half * L, v1row, tr, lane)
            pos = t * KROWS + half * L
            outv[pl.ds(pos, L)] = outv[pl.ds(pos, L)] + tot

    for k in range(NBUF):
        issue(k, k)

    def step(tt, _):
        a = NBUF * tt
        for k in range(NBUF):
            wait(a + k, k)
            compute(a + k, k)

            @pl.when(a + k + NBUF < NCHUNK)
            def _i():
                issue(a + k + NBUF, k)
        return ()

    lax.fori_loop(0, NCHUNK // NBUF, step, ())
    for k in range(NCHUNK % NBUF):
        t = (NCHUNK // NBUF) * NBUF + k
        wait(t, k)
        compute(t, k)
    for r in range(RPW):
        pltpu.sync_copy(outv.at[pl.ds(r * N_OUT, N_OUT)], out_hbm.at[b0 + r])


@jax.jit
def kernel(in_values, active_in_indices, active_label_indices, W1, b1, W2, b2):
    idx1 = active_in_indices.astype(jnp.int32).reshape(B * N_IN)
    idx2 = active_label_indices.astype(jnp.int32).reshape(B * N_OUT)
    vals = in_values.reshape(B * N_IN)
    zeros = jnp.zeros((RPW * F,), jnp.float32)
    x = _build_x(vals, idx1, zeros)
    val1 = _layer1(x, W1, b1)
    val2 = _layer2(val1, idx2, W2, b2)
    return val2, active_label_indices


# speedup vs baseline: 1.1227x; 1.1227x over previous
import functools

import jax
import jax.numpy as jnp
from jax import lax
from jax.experimental import pallas as pl
from jax.experimental.pallas import tpu as pltpu
from jax.experimental.pallas import tpu_sc as plsc

B = 256
N_IN = 64
N_OUT = 256
H = 1024
F = 8192
C = 32768

NC = 2
NS = 16
NW = NC * NS
RPW = B // NW
L = 16

_MESH = plsc.VectorSubcoreMesh(
    core_axis_name="c", subcore_axis_name="s", num_cores=NC, num_subcores=NS)


def _wid():
    return lax.axis_index("s") * NC + lax.axis_index("c")


@functools.partial(
    pl.kernel,
    out_type=(jax.ShapeDtypeStruct((B, F), jnp.float32),
              jax.ShapeDtypeStruct((B * N_OUT,), jnp.float32)),
    mesh=_MESH,
    compiler_params=pltpu.CompilerParams(needs_layout_passes=False),
    scratch_types=[
        pltpu.VMEM((RPW * F,), jnp.float32),
        pltpu.VMEM((RPW * N_IN,), jnp.int32),
        pltpu.VMEM((RPW * N_IN,), jnp.float32),
        pltpu.VMEM((RPW * N_OUT,), jnp.int32),
        pltpu.VMEM((RPW * N_OUT,), jnp.float32),
        pltpu.SemaphoreType.DMA,
    ],
)
def _build_x(vals_hbm, idx_hbm, zeros_hbm, idx2_hbm, b2_hbm,
             x_hbm, b2g_hbm, xbuf, idxv, valv, idx2v, b2gv, semb):
    w = _wid()
    b0 = w * RPW
    pltpu.sync_copy(idx2_hbm.at[pl.ds(b0 * N_OUT, RPW * N_OUT)], idx2v)
    for h in range(RPW * N_OUT // 128):
        pltpu.async_copy(b2_hbm.at[idx2v.at[pl.ds(h * 128, 128)]],
                         b2gv.at[pl.ds(h * 128, 128)], semb)
    pltpu.sync_copy(zeros_hbm, xbuf)
    pltpu.sync_copy(idx_hbm.at[pl.ds(b0 * N_IN, RPW * N_IN)], idxv)
    pltpu.sync_copy(vals_hbm.at[pl.ds(b0 * N_IN, RPW * N_IN)], valv)
    lane = lax.iota(jnp.int32, L)
    masks = [lane == l for l in range(L)]
    for r in range(RPW):
        for g in range(N_IN // L):
            ig = idxv[pl.ds(r * N_IN + g * L, L)] + (r * F)
            vg = valv[pl.ds(r * N_IN + g * L, L)]
            for l in range(L):
                plsc.addupdate_scatter(xbuf, [ig], vg, mask=masks[l])
    for r in range(RPW):
        pltpu.sync_copy(xbuf.at[pl.ds(r * F, F)], x_hbm.at[b0 + r])
    for h in range(RPW * N_OUT // 128):
        pltpu.make_async_copy(b2_hbm.at[idx2v.at[pl.ds(h * 128, 128)]],
                              b2gv.at[pl.ds(h * 128, 128)], semb).wait()
    pltpu.sync_copy(b2gv, b2g_hbm.at[pl.ds(b0 * N_OUT, RPW * N_OUT)])


_KBLK = F // 8


def _mm_body(x_ref, w_ref, b_ref, o_ref):
    k = pl.program_id(0)

    @pl.when(k == 0)
    def _init():
        o_ref[...] = jnp.zeros_like(o_ref)

    o_ref[...] += lax.dot_general(
        x_ref[...], w_ref[...], (((1,), (1,)), ((), ())),
        preferred_element_type=jnp.float32)

    @pl.when(k == pl.num_programs(0) - 1)
    def _fin():
        o_ref[...] = jnp.maximum(o_ref[...] + b_ref[...], 0.0)


def _layer1(x, w1, b1):
    return pl.pallas_call(
        _mm_body,
        grid=(F // _KBLK,),
        in_specs=[
            pl.BlockSpec((B, _KBLK), lambda k: (0, k)),
            pl.BlockSpec((H, _KBLK), lambda k: (0, k)),
            pl.BlockSpec((1, H), lambda k: (0, 0)),
        ],
        out_specs=pl.BlockSpec((B, H), lambda k: (0, 0)),
        out_shape=jax.ShapeDtypeStruct((B, H), jnp.float32),
    )(x, w1, b1[None, :])


KROWS = 16
NBUF = 4
NCHUNK = RPW * N_OUT // KROWS
CPB = N_OUT // KROWS


_UNROLL = 4


def _dot16(rows, row_off, v1row, tr, lane):
    def jstep(j, accs):
        for u in range(_UNROLL):
            jj = j * _UNROLL + u
            v1 = v1row[pl.ds(jj * L, L)]
            accs = tuple(accs[o] + rows[row_off + o, pl.ds(jj * L, L)] * v1
                         for o in range(L))
        return accs

    accs = lax.fori_loop(
        0, H // L // _UNROLL, jstep,
        tuple(jnp.zeros((L,), jnp.float32) for _ in range(L)))
    for o in range(L):
        tr[pl.ds(o * L, L)] = accs[o]
    tot = jnp.zeros((L,), jnp.float32)
    for l in range(L):
        col = plsc.load_gather(tr, [lane * L + l])
        tot = tot + col
    return tot


@functools.partial(
    pl.kernel,
    out_type=jax.ShapeDtypeStruct((B, N_OUT), jnp.float32),
    mesh=_MESH,
    compiler_params=pltpu.CompilerParams(needs_layout_passes=False),
    scratch_types=[
        pltpu.VMEM((RPW * N_OUT,), jnp.int32),
        pltpu.VMEM((H,), jnp.float32),
        pltpu.VMEM((NBUF, KROWS, H), jnp.float32),
        pltpu.VMEM((RPW * N_OUT,), jnp.float32),
        pltpu.VMEM((L * L,), jnp.float32),
    ] + [pltpu.SemaphoreType.DMA] * NBUF,
)
def _layer2(v1_hbm, idx_hbm, w2_hbm, b2g_hbm, out_hbm,
            idxv, v1row, ring, outv, tr, *sems):
    w = _wid()
    b0 = w * RPW
    lane = lax.iota(jnp.int32, L)
    pltpu.sync_copy(idx_hbm.at[pl.ds(b0 * N_OUT, RPW * N_OUT)], idxv)
    pltpu.sync_copy(b2g_hbm.at[pl.ds(b0 * N_OUT, RPW * N_OUT)], outv)

    def chunk_idx(t):
        return idxv.at[pl.ds(t * KROWS, KROWS)]

    def issue(t, k):
        return pltpu.async_copy(w2_hbm.at[chunk_idx(t)], ring.at[k], sems[k])

    def wait(t, k):
        pltpu.make_async_copy(w2_hbm.at[chunk_idx(t)], ring.at[k],
                              sems[k]).wait()

    def compute(t, k):
        @pl.when(t % CPB == 0)
        def _refresh():
            pltpu.sync_copy(v1_hbm.at[b0 + t // CPB], v1row)
        tot = _dot16(ring.at[k], 0, v1row, tr, lane)
        pos = t * KROWS
        outv[pl.ds(pos, L)] = outv[pl.ds(pos, L)] + tot

    for k in range(NBUF):
        issue(k, k)

    def step(tt, _):
        a = NBUF * tt
        for k in range(NBUF):
            wait(a + k, k)
            compute(a + k, k)

            @pl.when(a + k + NBUF < NCHUNK)
            def _i():
                issue(a + k + NBUF, k)
        return ()

    lax.fori_loop(0, NCHUNK // NBUF, step, ())
    for k in range(NCHUNK % NBUF):
        t = (NCHUNK // NBUF) * NBUF + k
        wait(t, k)
        compute(t, k)
    for r in range(RPW):
        pltpu.sync_copy(outv.at[pl.ds(r * N_OUT, N_OUT)], out_hbm.at[b0 + r])


@jax.jit
def kernel(in_values, active_in_indices, active_label_indices, W1, b1, W2, b2):
    idx1 = active_in_indices.astype(jnp.int32).reshape(B * N_IN)
    idx2 = active_label_indices.astype(jnp.int32).reshape(B * N_OUT)
    vals = in_values.reshape(B * N_IN)
    zeros = jnp.zeros((RPW * F,), jnp.float32)
    x, b2g = _build_x(vals, idx1, zeros, idx2, b2)
    val1 = _layer1(x, W1, b1)
    val2 = _layer2(val1, idx2, W2, b2g)
    return val2, active_label_indices


# speedup vs baseline: 1.1249x; 1.0020x over previous
import functools

import jax
import jax.numpy as jnp
from jax import lax
from jax.experimental import pallas as pl
from jax.experimental.pallas import tpu as pltpu
from jax.experimental.pallas import tpu_sc as plsc

B = 256
N_IN = 64
N_OUT = 256
H = 1024
F = 8192
C = 32768

NC = 2
NS = 16
NW = NC * NS
RPW = B // NW
L = 16

_MESH = plsc.VectorSubcoreMesh(
    core_axis_name="c", subcore_axis_name="s", num_cores=NC, num_subcores=NS)


def _wid():
    return lax.axis_index("s") * NC + lax.axis_index("c")


@functools.partial(
    pl.kernel,
    out_type=(jax.ShapeDtypeStruct((B, F), jnp.float32),
              jax.ShapeDtypeStruct((B * N_OUT,), jnp.float32)),
    mesh=_MESH,
    compiler_params=pltpu.CompilerParams(needs_layout_passes=False),
    scratch_types=[
        pltpu.VMEM((RPW * F,), jnp.float32),
        pltpu.VMEM((RPW * N_IN,), jnp.int32),
        pltpu.VMEM((RPW * N_IN,), jnp.float32),
        pltpu.VMEM((RPW * N_OUT,), jnp.int32),
        pltpu.VMEM((RPW * N_OUT,), jnp.float32),
        pltpu.SemaphoreType.DMA,
        pltpu.SemaphoreType.DMA,
    ],
)
def _build_x(vals_hbm, idx_hbm, zeros_hbm, idx2_hbm, b2_hbm,
             x_hbm, b2g_hbm, xbuf, idxv, valv, idx2v, b2gv, semb, semx):
    w = _wid()
    b0 = w * RPW
    pltpu.sync_copy(idx2_hbm.at[pl.ds(b0 * N_OUT, RPW * N_OUT)], idx2v)
    for h in range(RPW * N_OUT // 128):
        pltpu.async_copy(b2_hbm.at[idx2v.at[pl.ds(h * 128, 128)]],
                         b2gv.at[pl.ds(h * 128, 128)], semb)
    pltpu.sync_copy(zeros_hbm, xbuf)
    pltpu.sync_copy(idx_hbm.at[pl.ds(b0 * N_IN, RPW * N_IN)], idxv)
    pltpu.sync_copy(vals_hbm.at[pl.ds(b0 * N_IN, RPW * N_IN)], valv)
    lane = lax.iota(jnp.int32, L)
    masks = [lane == l for l in range(L)]
    for r in range(RPW):
        for g in range(N_IN // L):
            ig = idxv[pl.ds(r * N_IN + g * L, L)] + (r * F)
            vg = valv[pl.ds(r * N_IN + g * L, L)]
            for l in range(L):
                plsc.addupdate_scatter(xbuf, [ig], vg, mask=masks[l])
    for r in range(RPW):
        pltpu.async_copy(xbuf.at[pl.ds(r * F, F)], x_hbm.at[b0 + r], semx)
    for h in range(RPW * N_OUT // 128):
        pltpu.make_async_copy(b2_hbm.at[idx2v.at[pl.ds(h * 128, 128)]],
                              b2gv.at[pl.ds(h * 128, 128)], semb).wait()
    pltpu.sync_copy(b2gv, b2g_hbm.at[pl.ds(b0 * N_OUT, RPW * N_OUT)])
    for r in range(RPW):
        pltpu.make_async_copy(xbuf.at[pl.ds(r * F, F)], x_hbm.at[b0 + r],
                              semx).wait()


_KBLK = F // 8


def _mm_body(x_ref, w_ref, b_ref, o_ref):
    k = pl.program_id(0)

    @pl.when(k == 0)
    def _init():
        o_ref[...] = jnp.zeros_like(o_ref)

    o_ref[...] += lax.dot_general(
        x_ref[...], w_ref[...], (((1,), (1,)), ((), ())),
        preferred_element_type=jnp.float32)

    @pl.when(k == pl.num_programs(0) - 1)
    def _fin():
        o_ref[...] = jnp.maximum(o_ref[...] + b_ref[...], 0.0)


def _layer1(x, w1, b1):
    return pl.pallas_call(
        _mm_body,
        grid=(F // _KBLK,),
        in_specs=[
            pl.BlockSpec((B, _KBLK), lambda k: (0, k)),
            pl.BlockSpec((H, _KBLK), lambda k: (0, k)),
            pl.BlockSpec((1, H), lambda k: (0, 0)),
        ],
        out_specs=pl.BlockSpec((B, H), lambda k: (0, 0)),
        out_shape=jax.ShapeDtypeStruct((B, H), jnp.float32),
    )(x, w1, b1[None, :])


KROWS = 16
NBUF = 4
NCHUNK = RPW * N_OUT // KROWS
CPB = N_OUT // KROWS


_UNROLL = 4


def _dot16(rows, row_off, v1row, tr, lane):
    def jstep(j, accs):
        for u in range(_UNROLL):
            jj = j * _UNROLL + u
            v1 = v1row[pl.ds(jj * L, L)]
            accs = tuple(accs[o] + rows[row_off + o, pl.ds(jj * L, L)] * v1
                         for o in range(L))
        return accs

    accs = lax.fori_loop(
        0, H // L // _UNROLL, jstep,
        tuple(jnp.zeros((L,), jnp.float32) for _ in range(L)))
    for o in range(L):
        tr[pl.ds(o * L, L)] = accs[o]
    tot = jnp.zeros((L,), jnp.float32)
    for l in range(L):
        col = plsc.load_gather(tr, [lane * L + l])
        tot = tot + col
    return tot


@functools.partial(
    pl.kernel,
    out_type=jax.ShapeDtypeStruct((B, N_OUT), jnp.float32),
    mesh=_MESH,
    compiler_params=pltpu.CompilerParams(needs_layout_passes=False),
    scratch_types=[
        pltpu.VMEM((RPW * N_OUT,), jnp.int32),
        pltpu.VMEM((H,), jnp.float32),
        pltpu.VMEM((NBUF, KROWS, H), jnp.float32),
        pltpu.VMEM((RPW * N_OUT,), jnp.float32),
        pltpu.VMEM((L * L,), jnp.float32),
    ] + [pltpu.SemaphoreType.DMA] * NBUF,
)
def _layer2(v1_hbm, idx_hbm, w2_hbm, b2g_hbm, out_hbm,
            idxv, v1row, ring, outv, tr, *sems):
    w = _wid()
    b0 = w * RPW
    lane = lax.iota(jnp.int32, L)
    pltpu.sync_copy(idx_hbm.at[pl.ds(b0 * N_OUT, RPW * N_OUT)], idxv)
    pltpu.sync_copy(b2g_hbm.at[pl.ds(b0 * N_OUT, RPW * N_OUT)], outv)

    def chunk_idx(t):
        return idxv.at[pl.ds(t * KROWS, KROWS)]

    def issue(t, k):
        return pltpu.async_copy(w2_hbm.at[chunk_idx(t)], ring.at[k], sems[k])

    def wait(t, k):
        pltpu.make_async_copy(w2_hbm.at[chunk_idx(t)], ring.at[k],
                              sems[k]).wait()

    def compute(t, k):
        @pl.when(t % CPB == 0)
        def _refresh():
            pltpu.sync_copy(v1_hbm.at[b0 + t // CPB], v1row)
        tot = _dot16(ring.at[k], 0, v1row, tr, lane)
        pos = t * KROWS
        outv[pl.ds(pos, L)] = outv[pl.ds(pos, L)] + tot

    for k in range(NBUF):
        issue(k, k)

    def step(tt, _):
        a = NBUF * tt
        for k in range(NBUF):
            wait(a + k, k)
            compute(a + k, k)

            @pl.when(a + k + NBUF < NCHUNK)
            def _i():
                issue(a + k + NBUF, k)
        return ()

    lax.fori_loop(0, NCHUNK // NBUF, step, ())
    for k in range(NCHUNK % NBUF):
        t = (NCHUNK // NBUF) * NBUF + k
        wait(t, k)
        compute(t, k)
    for r in range(RPW):
        pltpu.sync_copy(outv.at[pl.ds(r * N_OUT, N_OUT)], out_hbm.at[b0 + r])


@jax.jit
def kernel(in_values, active_in_indices, active_label_indices, W1, b1, W2, b2):
    idx1 = active_in_indices.astype(jnp.int32).reshape(B * N_IN)
    idx2 = active_label_indices.astype(jnp.int32).reshape(B * N_OUT)
    vals = in_values.reshape(B * N_IN)
    zeros = jnp.zeros((RPW * F,), jnp.float32)
    x, b2g = _build_x(vals, idx1, zeros, idx2, b2)
    val1 = _layer1(x, W1, b1)
    val2 = _layer2(val1, idx2, W2, b2g)
    return val2, active_label_indices


# speedup vs baseline: 1.1279x; 1.0026x over previous
import functools

import jax
import jax.numpy as jnp
from jax import lax
from jax.experimental import pallas as pl
from jax.experimental.pallas import tpu as pltpu
from jax.experimental.pallas import tpu_sc as plsc

B = 256
N_IN = 64
N_OUT = 256
H = 1024
F = 8192
C = 32768

NC = 2
NS = 16
NW = NC * NS
RPW = B // NW
L = 16

_MESH = plsc.VectorSubcoreMesh(
    core_axis_name="c", subcore_axis_name="s", num_cores=NC, num_subcores=NS)


def _wid():
    return lax.axis_index("s") * NC + lax.axis_index("c")


@functools.partial(
    pl.kernel,
    out_type=(jax.ShapeDtypeStruct((B, F), jnp.float32),
              jax.ShapeDtypeStruct((B * N_OUT,), jnp.float32)),
    mesh=_MESH,
    compiler_params=pltpu.CompilerParams(needs_layout_passes=False),
    scratch_types=[
        pltpu.VMEM((RPW * F,), jnp.float32),
        pltpu.VMEM((RPW * N_IN,), jnp.int32),
        pltpu.VMEM((RPW * N_IN,), jnp.float32),
        pltpu.VMEM((RPW * N_OUT,), jnp.int32),
        pltpu.VMEM((RPW * N_OUT,), jnp.float32),
        pltpu.SemaphoreType.DMA,
        pltpu.SemaphoreType.DMA,
    ],
)
def _build_x(vals_hbm, idx_hbm, zeros_hbm, idx2_hbm, b2_hbm,
             x_hbm, b2g_hbm, xbuf, idxv, valv, idx2v, b2gv, semb, semx):
    w = _wid()
    b0 = w * RPW
    pltpu.sync_copy(idx2_hbm.at[pl.ds(b0 * N_OUT, RPW * N_OUT)], idx2v)
    for h in range(RPW * N_OUT // 128):
        pltpu.async_copy(b2_hbm.at[idx2v.at[pl.ds(h * 128, 128)]],
                         b2gv.at[pl.ds(h * 128, 128)], semb)
    pltpu.sync_copy(zeros_hbm, xbuf)
    pltpu.sync_copy(idx_hbm.at[pl.ds(b0 * N_IN, RPW * N_IN)], idxv)
    pltpu.sync_copy(vals_hbm.at[pl.ds(b0 * N_IN, RPW * N_IN)], valv)
    lane = lax.iota(jnp.int32, L)
    masks = [lane == l for l in range(L)]
    for r in range(RPW):
        for g in range(N_IN // L):
            ig = idxv[pl.ds(r * N_IN + g * L, L)] + (r * F)
            vg = valv[pl.ds(r * N_IN + g * L, L)]
            for l in range(L):
                plsc.addupdate_scatter(xbuf, [ig], vg, mask=masks[l])
    for r in range(RPW):
        pltpu.async_copy(xbuf.at[pl.ds(r * F, F)], x_hbm.at[b0 + r], semx)
    for h in range(RPW * N_OUT // 128):
        pltpu.make_async_copy(b2_hbm.at[idx2v.at[pl.ds(h * 128, 128)]],
                              b2gv.at[pl.ds(h * 128, 128)], semb).wait()
    pltpu.sync_copy(b2gv, b2g_hbm.at[pl.ds(b0 * N_OUT, RPW * N_OUT)])
    for r in range(RPW):
        pltpu.make_async_copy(xbuf.at[pl.ds(r * F, F)], x_hbm.at[b0 + r],
                              semx).wait()


_KBLK = F // 4


def _mm_body(x_ref, w_ref, b_ref, o_ref):
    k = pl.program_id(0)

    @pl.when(k == 0)
    def _init():
        o_ref[...] = jnp.zeros_like(o_ref)

    o_ref[...] += lax.dot_general(
        x_ref[...], w_ref[...], (((1,), (1,)), ((), ())),
        preferred_element_type=jnp.float32)

    @pl.when(k == pl.num_programs(0) - 1)
    def _fin():
        o_ref[...] = jnp.maximum(o_ref[...] + b_ref[...], 0.0)


def _layer1(x, w1, b1):
    return pl.pallas_call(
        _mm_body,
        grid=(F // _KBLK,),
        in_specs=[
            pl.BlockSpec((B, _KBLK), lambda k: (0, k)),
            pl.BlockSpec((H, _KBLK), lambda k: (0, k)),
            pl.BlockSpec((1, H), lambda k: (0, 0)),
        ],
        out_specs=pl.BlockSpec((B, H), lambda k: (0, 0)),
        out_shape=jax.ShapeDtypeStruct((B, H), jnp.float32),
    )(x, w1, b1[None, :])


KROWS = 16
NBUF = 4
NCHUNK = RPW * N_OUT // KROWS
CPB = N_OUT // KROWS


_UNROLL = 4


def _dot16(rows, row_off, v1row, tr, lane):
    def jstep(j, accs):
        for u in range(_UNROLL):
            jj = j * _UNROLL + u
            v1 = v1row[pl.ds(jj * L, L)]
            accs = tuple(accs[o] + rows[row_off + o, pl.ds(jj * L, L)] * v1
                         for o in range(L))
        return accs

    accs = lax.fori_loop(
        0, H // L // _UNROLL, jstep,
        tuple(jnp.zeros((L,), jnp.float32) for _ in range(L)))
    for o in range(L):
        tr[pl.ds(o * L, L)] = accs[o]
    tot = jnp.zeros((L,), jnp.float32)
    for l in range(L):
        col = plsc.load_gather(tr, [lane * L + l])
        tot = tot + col
    return tot


@functools.partial(
    pl.kernel,
    out_type=jax.ShapeDtypeStruct((B, N_OUT), jnp.float32),
    mesh=_MESH,
    compiler_params=pltpu.CompilerParams(needs_layout_passes=False),
    scratch_types=[
        pltpu.VMEM((RPW * N_OUT,), jnp.int32),
        pltpu.VMEM((H,), jnp.float32),
        pltpu.VMEM((NBUF, KROWS, H), jnp.float32),
        pltpu.VMEM((RPW * N_OUT,), jnp.float32),
        pltpu.VMEM((L * L,), jnp.float32),
    ] + [pltpu.SemaphoreType.DMA] * NBUF,
)
def _layer2(v1_hbm, idx_hbm, w2_hbm, b2g_hbm, out_hbm,
            idxv, v1row, ring, outv, tr, *sems):
    w = _wid()
    b0 = w * RPW
    lane = lax.iota(jnp.int32, L)
    pltpu.sync_copy(idx_hbm.at[pl.ds(b0 * N_OUT, RPW * N_OUT)], idxv)
    pltpu.sync_copy(b2g_hbm.at[pl.ds(b0 * N_OUT, RPW * N_OUT)], outv)

    def chunk_idx(t):
        return idxv.at[pl.ds(t * KROWS, KROWS)]

    def issue(t, k):
        return pltpu.async_copy(w2_hbm.at[chunk_idx(t)], ring.at[k], sems[k])

    def wait(t, k):
        pltpu.make_async_copy(w2_hbm.at[chunk_idx(t)], ring.at[k],
                              sems[k]).wait()

    def compute(t, k):
        @pl.when(t % CPB == 0)
        def _refresh():
            pltpu.sync_copy(v1_hbm.at[b0 + t // CPB], v1row)
        tot = _dot16(ring.at[k], 0, v1row, tr, lane)
        pos = t * KROWS
        outv[pl.ds(pos, L)] = outv[pl.ds(pos, L)] + tot

    for k in range(NBUF):
        issue(k, k)

    def step(tt, _):
        a = NBUF * tt
        for k in range(NBUF):
            wait(a + k, k)
            compute(a + k, k)

            @pl.when(a + k + NBUF < NCHUNK)
            def _i():
                issue(a + k + NBUF, k)
        return ()

    lax.fori_loop(0, NCHUNK // NBUF, step, ())
    for k in range(NCHUNK % NBUF):
        t = (NCHUNK // NBUF) * NBUF + k
        wait(t, k)
        compute(t, k)
    for r in range(RPW):
        pltpu.sync_copy(outv.at[pl.ds(r * N_OUT, N_OUT)], out_hbm.at[b0 + r])


@jax.jit
def kernel(in_values, active_in_indices, active_label_indices, W1, b1, W2, b2):
    idx1 = active_in_indices.astype(jnp.int32).reshape(B * N_IN)
    idx2 = active_label_indices.astype(jnp.int32).reshape(B * N_OUT)
    vals = in_values.reshape(B * N_IN)
    zeros = jnp.zeros((RPW * F,), jnp.float32)
    x, b2g = _build_x(vals, idx1, zeros, idx2, b2)
    val1 = _layer1(x, W1, b1)
    val2 = _layer2(val1, idx2, W2, b2g)
    return val2, active_label_indices
